# baseline (speedup 1.0000x reference)
import jax
import jax.numpy as jnp
from jax.experimental import pallas as pl

_F_BLOCK = 1024


def _ffn_kernel(se_ref, rw_ref, x_ref, w1_ref, w2_ref, o_ref):
    e = pl.program_id(0)
    fb = pl.program_id(1)

    fstart = pl.multiple_of(fb * _F_BLOCK, _F_BLOCK)
    w1_blk = w1_ref[0, :, pl.ds(fstart, _F_BLOCK)]
    h = jnp.dot(x_ref[:, :], w1_blk, preferred_element_type=jnp.float32)
    h = h * jax.nn.sigmoid(h)
    o = jnp.dot(h, w2_ref[0], preferred_element_type=jnp.float32)

    mask = (se_ref[:, :] == e).astype(jnp.float32)
    coef = jnp.sum(rw_ref[:, :] * mask, axis=1)
    contrib = o * coef[:, None]

    @pl.when(jnp.logical_and(e == 0, fb == 0))
    def _init():
        o_ref[:, :] = jnp.zeros_like(o_ref)

    o_ref[:, :] += contrib


def kernel(x, routing_weights, selected_experts, w1, w2):
    shape = x.shape
    D = shape[-1]
    K = routing_weights.shape[-1]
    x_flat = x.reshape(-1, D)
    rw_flat = routing_weights.reshape(-1, K).astype(jnp.float32)
    se_flat = selected_experts.reshape(-1, K).astype(jnp.int32)
    S = x_flat.shape[0]
    E, _, F = w1.shape
    nf = F // _F_BLOCK

    out = pl.pallas_call(
        _ffn_kernel,
        grid=(E, nf),
        in_specs=[
            pl.BlockSpec((S, K), lambda e, fb: (0, 0)),
            pl.BlockSpec((S, K), lambda e, fb: (0, 0)),
            pl.BlockSpec((S, D), lambda e, fb: (0, 0)),
            pl.BlockSpec((1, D, F), lambda e, fb: (e, 0, 0)),
            pl.BlockSpec((1, _F_BLOCK, D), lambda e, fb: (e, fb, 0)),
        ],
        out_specs=pl.BlockSpec((S, D), lambda e, fb: (0, 0)),
        out_shape=jax.ShapeDtypeStruct((S, D), jnp.float32),
    )(se_flat, rw_flat, x_flat, w1, w2)

    return out.reshape(shape)


# speedup vs baseline: 1.1075x; 1.1075x over previous
import jax
import jax.numpy as jnp
from jax.experimental import pallas as pl

_F_BLOCK = 1024


def _ffn_kernel(se_ref, rw_ref, x_ref, w1_ref, w2_ref, o_ref):
    e = pl.program_id(0)
    fb = pl.program_id(1)

    h = jnp.dot(x_ref[:, :], w1_ref[0], preferred_element_type=jnp.float32)
    h = h * jax.nn.sigmoid(h)
    o = jnp.dot(h, w2_ref[0], preferred_element_type=jnp.float32)

    mask = (se_ref[:, :] == e).astype(jnp.float32)
    coef = jnp.sum(rw_ref[:, :] * mask, axis=1)
    contrib = o * coef[:, None]

    @pl.when(jnp.logical_and(e == 0, fb == 0))
    def _init():
        o_ref[:, :] = jnp.zeros_like(o_ref)

    o_ref[:, :] += contrib


def kernel(x, routing_weights, selected_experts, w1, w2):
    shape = x.shape
    D = shape[-1]
    K = routing_weights.shape[-1]
    x_flat = x.reshape(-1, D)
    rw_flat = routing_weights.reshape(-1, K).astype(jnp.float32)
    se_flat = selected_experts.reshape(-1, K).astype(jnp.int32)
    S = x_flat.shape[0]
    E, _, F = w1.shape
    nf = F // _F_BLOCK

    out = pl.pallas_call(
        _ffn_kernel,
        grid=(E, nf),
        in_specs=[
            pl.BlockSpec((S, K), lambda e, fb: (0, 0)),
            pl.BlockSpec((S, K), lambda e, fb: (0, 0)),
            pl.BlockSpec((S, D), lambda e, fb: (0, 0)),
            pl.BlockSpec((1, D, _F_BLOCK), lambda e, fb: (e, 0, fb)),
            pl.BlockSpec((1, _F_BLOCK, D), lambda e, fb: (e, fb, 0)),
        ],
        out_specs=pl.BlockSpec((S, D), lambda e, fb: (0, 0)),
        out_shape=jax.ShapeDtypeStruct((S, D), jnp.float32),
    )(se_flat, rw_flat, x_flat, w1, w2)

    return out.reshape(shape)
